# baseline (speedup 1.0000x reference)
import functools

import jax
import jax.numpy as jnp
from jax import lax
from jax.experimental import pallas as pl
from jax.experimental.pallas import tpu as pltpu
from jax.experimental.pallas import tpu_sc as plsc

NUM_TABLES = 26
TABLE_ROWS = 100000
DIM = 32
BATCH = 16384

CHUNK = 256
BLOCKS = BATCH // CHUNK
FLAT = BATCH * NUM_TABLES
L = 16
ELEMS = DIM * CHUNK


def _make_kernel(num_workers):
    bpw = BLOCKS // num_workers
    n_chunks = bpw * NUM_TABLES
    mesh = plsc.VectorSubcoreMesh(core_axis_name="c", subcore_axis_name="s")

    @functools.partial(
        pl.kernel,
        mesh=mesh,
        out_type=jax.ShapeDtypeStruct((NUM_TABLES * DIM, BATCH), jnp.float32),
        scratch_types=[
            pltpu.VMEM((n_chunks, CHUNK), jnp.int32),
            pltpu.VMEM((2, ELEMS), jnp.int32),
            pltpu.VMEM((2, ELEMS), jnp.float32),
            pltpu.VMEM((2, DIM, CHUNK), jnp.float32),
            pltpu.SemaphoreType.DMA((2,)),
            pltpu.SemaphoreType.DMA((2,)),
        ],
        compiler_params=pltpu.CompilerParams(use_tc_tiling_on_sc=False),
    )
    def gather_kernel(tab_hbm, idx_hbm, out_hbm, idx_v, el_v, rows_v, slab_v,
                      gsem, ssem):
        wid = lax.axis_index("s") * 2 + lax.axis_index("c")
        base_b = wid * bpw * CHUNK

        def stage(t, _):
            pltpu.sync_copy(
                idx_hbm.at[pl.ds(t * BLOCKS + wid * bpw, bpw)],
                idx_v.at[pl.ds(t * bpw, bpw)])
            return 0

        lax.fori_loop(0, NUM_TABLES, stage, 0)

        def build_idx(j, buf):
            t = j // bpw
            for s in range(CHUNK // L):
                fused = idx_v[j, pl.ds(s * L, L)] + t * TABLE_ROWS
                base = (fused >> 7) * 1024 + (fused & 127)
                for c in range(DIM):
                    kc = (c // 8) * (TILECOLS * 1024) + (c % 8) * 128
                    el_v[buf, pl.ds(c * CHUNK + s * L, L)] = base + kc
            return

        def fire_gather(buf):
            pltpu.async_copy(tab_hbm.at[el_v.at[buf]], rows_v.at[buf],
                             gsem.at[buf])

        def wait_gather(buf):
            pltpu.make_async_copy(tab_hbm.at[el_v.at[buf]], rows_v.at[buf],
                                  gsem.at[buf]).wait()

        def repack(buf):
            for c in range(DIM):
                for s in range(CHUNK // L):
                    slab_v[buf, c, pl.ds(s * L, L)] = (
                        rows_v[buf, pl.ds(c * CHUNK + s * L, L)])

        def out_slab(j):
            t = j // bpw
            bb = j % bpw
            return out_hbm.at[pl.ds(t * DIM, DIM),
                              pl.ds(base_b + bb * CHUNK, CHUNK)]

        def wait_store(buf):
            pltpu.make_async_copy(slab_v.at[buf],
                                  out_hbm.at[pl.ds(0, DIM), pl.ds(0, CHUNK)],
                                  ssem.at[buf]).wait()

        build_idx(0, 0)
        fire_gather(0)

        def step(p, _):
            for buf in (0, 1):
                j = 2 * p + buf

                @pl.when(j + 1 < n_chunks)
                def _():
                    build_idx(j + 1, 1 - buf)

                @pl.when(j >= 1)
                def _():
                    wait_store(1 - buf)

                @pl.when(j + 1 < n_chunks)
                def _():
                    fire_gather(1 - buf)

                wait_gather(buf)
                repack(buf)
                pltpu.async_copy(slab_v.at[buf], out_slab(j), ssem.at[buf])
            return 0

        lax.fori_loop(0, n_chunks // 2, step, 0)
        wait_store(1)

    return gather_kernel


FLAT_ROWS = TABLE_ROWS * NUM_TABLES
TILECOLS = (FLAT_ROWS + 127) // 128

_kernel_32 = _make_kernel(32)


@jax.jit
def kernel(indices, embedding_table):
    idxt = lax.optimization_barrier(
        indices.astype(jnp.int32).T.reshape(-1))
    idx = idxt.reshape(NUM_TABLES * BLOCKS, CHUNK)
    t2p = jnp.pad(embedding_table.T, ((0, 0), (0, TILECOLS * 128 - FLAT_ROWS)))
    tflat = t2p.reshape(4, 8, TILECOLS, 128).transpose(0, 2, 1, 3).reshape(-1)
    v = _kernel_32(tflat, idx)
    v3 = v.reshape(NUM_TABLES, DIM, BATCH)
    return jnp.transpose(v3, (2, 0, 1))


# speedup vs baseline: 1.0082x; 1.0082x over previous
import functools

import jax
import jax.numpy as jnp
from jax import lax
from jax.experimental import pallas as pl
from jax.experimental.pallas import tpu as pltpu
from jax.experimental.pallas import tpu_sc as plsc

NUM_TABLES = 26
TABLE_ROWS = 100000
DIM = 32
BATCH = 16384

CHUNK = 128
BLOCKS = BATCH // CHUNK
FLAT = BATCH * NUM_TABLES
L = 16
ELEMS = DIM * CHUNK


def _make_kernel(num_workers):
    bpw = BLOCKS // num_workers
    n_chunks = bpw * NUM_TABLES
    mesh = plsc.VectorSubcoreMesh(core_axis_name="c", subcore_axis_name="s")

    @functools.partial(
        pl.kernel,
        mesh=mesh,
        out_type=jax.ShapeDtypeStruct((NUM_TABLES * DIM, BATCH), jnp.float32),
        scratch_types=[
            pltpu.VMEM((n_chunks, CHUNK), jnp.int32),
            pltpu.VMEM((2, ELEMS), jnp.int32),
            pltpu.VMEM((2, ELEMS), jnp.float32),
            pltpu.VMEM((2, DIM, CHUNK), jnp.float32),
            pltpu.SemaphoreType.DMA((2,)),
            pltpu.SemaphoreType.DMA((2,)),
        ],
        compiler_params=pltpu.CompilerParams(use_tc_tiling_on_sc=False),
    )
    def gather_kernel(tab_hbm, idx_hbm, out_hbm, idx_v, el_v, rows_v, slab_v,
                      gsem, ssem):
        wid = lax.axis_index("s") * 2 + lax.axis_index("c")
        base_b = wid * bpw * CHUNK

        def stage(t, _):
            pltpu.sync_copy(
                idx_hbm.at[pl.ds(t * BLOCKS + wid * bpw, bpw)],
                idx_v.at[pl.ds(t * bpw, bpw)])
            return 0

        lax.fori_loop(0, NUM_TABLES, stage, 0)

        def build_idx(j, buf):
            t = j // bpw
            for s in range(CHUNK // L):
                fused = idx_v[j, pl.ds(s * L, L)] + t * TABLE_ROWS
                base = (fused >> 7) * 1024 + (fused & 127)
                for c in range(DIM):
                    kc = (c // 8) * (TILECOLS * 1024) + (c % 8) * 128
                    el_v[buf, pl.ds(c * CHUNK + s * L, L)] = base + kc
            return

        def fire_gather(buf):
            pltpu.async_copy(tab_hbm.at[el_v.at[buf]], rows_v.at[buf],
                             gsem.at[buf])

        def wait_gather(buf):
            pltpu.make_async_copy(tab_hbm.at[el_v.at[buf]], rows_v.at[buf],
                                  gsem.at[buf]).wait()

        def repack(buf):
            for c in range(DIM):
                for s in range(CHUNK // L):
                    slab_v[buf, c, pl.ds(s * L, L)] = (
                        rows_v[buf, pl.ds(c * CHUNK + s * L, L)])

        def out_slab(j):
            t = j // bpw
            bb = j % bpw
            return out_hbm.at[pl.ds(t * DIM, DIM),
                              pl.ds(base_b + bb * CHUNK, CHUNK)]

        def wait_store(buf):
            pltpu.make_async_copy(slab_v.at[buf],
                                  out_hbm.at[pl.ds(0, DIM), pl.ds(0, CHUNK)],
                                  ssem.at[buf]).wait()

        build_idx(0, 0)
        fire_gather(0)

        def step(p, _):
            for buf in (0, 1):
                j = 2 * p + buf

                @pl.when(j + 1 < n_chunks)
                def _():
                    build_idx(j + 1, 1 - buf)

                @pl.when(j >= 1)
                def _():
                    wait_store(1 - buf)

                @pl.when(j + 1 < n_chunks)
                def _():
                    fire_gather(1 - buf)

                wait_gather(buf)
                repack(buf)
                pltpu.async_copy(slab_v.at[buf], out_slab(j), ssem.at[buf])
            return 0

        lax.fori_loop(0, n_chunks // 2, step, 0)
        wait_store(1)

    return gather_kernel


FLAT_ROWS = TABLE_ROWS * NUM_TABLES
TILECOLS = (FLAT_ROWS + 127) // 128

_kernel_32 = _make_kernel(32)


@jax.jit
def kernel(indices, embedding_table):
    idxt = lax.optimization_barrier(
        indices.astype(jnp.int32).T.reshape(-1))
    idx = idxt.reshape(NUM_TABLES * BLOCKS, CHUNK)
    t2p = jnp.pad(embedding_table.T, ((0, 0), (0, TILECOLS * 128 - FLAT_ROWS)))
    tflat = t2p.reshape(4, 8, TILECOLS, 128).transpose(0, 2, 1, 3).reshape(-1)
    v = _kernel_32(tflat, idx)
    v3 = v.reshape(NUM_TABLES, DIM, BATCH)
    return jnp.transpose(v3, (2, 0, 1))
